# Initial kernel scaffold; baseline (speedup 1.0000x reference)
import jax
import jax.numpy as jnp
from jax.experimental import pallas as pl


def kernel(codon_ids, species_ids, mask, weight_matrix):
    raise NotImplementedError("write your pallas kernel here")


# speedup vs baseline: 260.7431x; 260.7431x over previous
import functools

import jax
import jax.numpy as jnp
from jax import lax
from jax.experimental import pallas as pl
from jax.experimental.pallas import tpu as pltpu
from jax.experimental.pallas import tpu_sc as plsc

N_SPECIES = 5
N_CODONS = 64
B = 4096
L = 2048
LW = L // 4

_info = plsc.get_sparse_core_info()
NC, NS, LANES = _info.num_cores, _info.num_subcores, _info.num_lanes
NW = NC * NS
RPW = B // NW
GROUPS = RPW // LANES


def _cai_sc(cid_hbm, sid_hbm, mw_hbm, logw_hbm, out_hbm,
            cid_v, mw_v, table_v, sid_v, out_v):
    wid = lax.axis_index("s") * NC + lax.axis_index("c")
    base_row = wid * RPW

    pltpu.sync_copy(logw_hbm, table_v)
    pltpu.sync_copy(sid_hbm.at[pl.ds(base_row, RPW)], sid_v)

    row16 = lax.iota(jnp.int32, LANES)
    cidx0 = row16 * L
    widx0 = row16 * LW

    for g in range(GROUPS):
        r0 = base_row + g * LANES
        pltpu.sync_copy(cid_hbm.at[pl.ds(r0 * L, LANES * L)], cid_v)
        pltpu.sync_copy(mw_hbm.at[pl.ds(r0 * LW, LANES * LW)], mw_v)

        sidv = sid_v[pl.ds(g * LANES, LANES)]
        sb = sidv * N_CODONS

        def body(l4, carry):
            acc, cnt, cidx, widx = carry
            w = plsc.load_gather(mw_v, [widx])
            for j in range(4):
                cid = plsc.load_gather(cid_v, [cidx])
                gval = plsc.load_gather(table_v, [cid + sb])
                m = (w >> (8 * j)) & 1 if j else w & 1
                acc = acc + gval * m.astype(jnp.float32)
                cnt = cnt + m
                cidx = cidx + 1
            return acc, cnt, cidx, widx + 1

        acc, cnt, _, _ = lax.fori_loop(
            0, LW,
            body,
            (jnp.zeros((LANES,), jnp.float32), jnp.zeros((LANES,), jnp.int32),
             cidx0, widx0),
        )

        cnt_f = jnp.maximum(cnt.astype(jnp.float32), 1.0)
        out_v[pl.ds(g * LANES, LANES)] = jnp.exp(acc / cnt_f)

    pltpu.sync_copy(out_v, out_hbm.at[pl.ds(base_row, RPW)])


@jax.jit
def kernel(codon_ids, species_ids, mask, weight_matrix):
    logw = jnp.log(jnp.maximum(weight_matrix, 1e-8)).reshape(-1)
    mask_words = lax.bitcast_convert_type(
        mask.astype(jnp.uint8).reshape(B, LW, 4), jnp.int32)

    mesh = plsc.VectorSubcoreMesh(core_axis_name="c", subcore_axis_name="s")
    run = pl.kernel(
        _cai_sc,
        mesh=mesh,
        compiler_params=pltpu.CompilerParams(needs_layout_passes=False),
        out_type=jax.ShapeDtypeStruct((B,), jnp.float32),
        scratch_types=[
            pltpu.VMEM((LANES * L,), jnp.int32),
            pltpu.VMEM((LANES * LW,), jnp.int32),
            pltpu.VMEM((N_SPECIES * N_CODONS,), jnp.float32),
            pltpu.VMEM((RPW,), jnp.int32),
            pltpu.VMEM((RPW,), jnp.float32),
        ],
    )
    return run(codon_ids.reshape(-1), species_ids, mask_words.reshape(-1), logw)


# speedup vs baseline: 415.6171x; 1.5940x over previous
import functools

import jax
import jax.numpy as jnp
from jax import lax
from jax.experimental import pallas as pl
from jax.experimental.pallas import tpu as pltpu
from jax.experimental.pallas import tpu_sc as plsc

N_SPECIES = 5
N_CODONS = 64
B = 4096
L = 2048
LW = L // 4

_info = plsc.get_sparse_core_info()
NC, NS, LANES = _info.num_cores, _info.num_subcores, _info.num_lanes
NW = NC * NS
RPW = B // NW
NCHUNK = 4
CROWS = RPW // NCHUNK
SUBG = CROWS // LANES


def _cai_sc(pk_hbm, sid_hbm, tbl_hbm, out_hbm,
            b0, b1, b2, b3, tbl_v, sid_v, out_v, s0, s1, s2, s3):
    wid = lax.axis_index("s") * NC + lax.axis_index("c")
    base_row = wid * RPW

    bufs = (b0, b1, b2, b3)
    sems = (s0, s1, s2, s3)
    copies = []
    for c in range(NCHUNK):
        off = (base_row + c * CROWS) * LW
        cp = pltpu.make_async_copy(
            pk_hbm.at[pl.ds(off, CROWS * LW)], bufs[c], sems[c])
        cp.start()
        copies.append(cp)

    pltpu.sync_copy(tbl_hbm, tbl_v)
    pltpu.sync_copy(sid_hbm.at[pl.ds(base_row, RPW)], sid_v)

    row16 = lax.iota(jnp.int32, LANES)

    for c in range(NCHUNK):
        copies[c].wait()
        for sub in range(SUBG):
            lg = c * SUBG + sub
            sidv = sid_v[pl.ds(lg * LANES, LANES)]
            sb = sidv * 128
            widx0 = row16 * LW + (sub * LANES * LW)

            def body(_, carry):
                acc, cnt, widx = carry
                w = plsc.load_gather(bufs[c], [widx])
                for j in range(4):
                    t = (w >> (8 * j)) if j else w
                    g = plsc.load_gather(tbl_v, [(t & 0x7F) + sb])
                    acc = acc + g
                    cnt = cnt + (t & 0x40)
                return acc, cnt, widx + 1

            acc, cnt, _ = lax.fori_loop(
                0, LW, body,
                (jnp.zeros((LANES,), jnp.float32),
                 jnp.zeros((LANES,), jnp.int32), widx0))

            cnt_f = jnp.maximum((cnt >> 6).astype(jnp.float32), 1.0)
            out_v[pl.ds(lg * LANES, LANES)] = jnp.exp(acc / cnt_f)

    pltpu.sync_copy(out_v, out_hbm.at[pl.ds(base_row, RPW)])


@jax.jit
def kernel(codon_ids, species_ids, mask, weight_matrix):
    logw = jnp.log(jnp.maximum(weight_matrix, 1e-8))
    tbl = jnp.concatenate(
        [jnp.zeros((N_SPECIES, N_CODONS), jnp.float32), logw], axis=1)

    packed = (codon_ids | (mask.astype(jnp.int32) << 6)).astype(jnp.uint8)
    packed_words = lax.bitcast_convert_type(
        packed.reshape(B, LW, 4), jnp.int32).reshape(-1)

    mesh = plsc.VectorSubcoreMesh(core_axis_name="c", subcore_axis_name="s")
    run = pl.kernel(
        _cai_sc,
        mesh=mesh,
        compiler_params=pltpu.CompilerParams(needs_layout_passes=False),
        out_type=jax.ShapeDtypeStruct((B,), jnp.float32),
        scratch_types=[
            pltpu.VMEM((CROWS * LW,), jnp.int32),
            pltpu.VMEM((CROWS * LW,), jnp.int32),
            pltpu.VMEM((CROWS * LW,), jnp.int32),
            pltpu.VMEM((CROWS * LW,), jnp.int32),
            pltpu.VMEM((N_SPECIES * 128,), jnp.float32),
            pltpu.VMEM((RPW,), jnp.int32),
            pltpu.VMEM((RPW,), jnp.float32),
            pltpu.SemaphoreType.DMA,
            pltpu.SemaphoreType.DMA,
            pltpu.SemaphoreType.DMA,
            pltpu.SemaphoreType.DMA,
        ],
    )
    return run(packed_words, species_ids, tbl.reshape(-1))


# speedup vs baseline: 699.5256x; 1.6831x over previous
import functools

import jax
import jax.numpy as jnp
from jax import lax
from jax.experimental import pallas as pl
from jax.experimental.pallas import tpu as pltpu
from jax.experimental.pallas import tpu_sc as plsc

N_SPECIES = 5
N_CODONS = 64
B = 4096
L = 2048
LW = L // 4

_info = plsc.get_sparse_core_info()
NC, NS, LANES = _info.num_cores, _info.num_subcores, _info.num_lanes
NW = NC * NS
RPW = B // NW
NCHUNK = 4
CROWS = RPW // NCHUNK
SUBG = CROWS // LANES


def _cai_sc(pk_hbm, sid_hbm, tbl_hbm, out_hbm,
            b0, b1, b2, b3, tbl_v, sid_v, out_v, s0, s1, s2, s3):
    wid = lax.axis_index("s") * NC + lax.axis_index("c")
    base_row = wid * RPW

    bufs = (b0, b1, b2, b3)
    sems = (s0, s1, s2, s3)
    copies = []
    for c in range(NCHUNK):
        off = (base_row + c * CROWS) * LW
        cp = pltpu.make_async_copy(
            pk_hbm.at[pl.ds(off, CROWS * LW)], bufs[c], sems[c])
        cp.start()
        copies.append(cp)

    pltpu.sync_copy(tbl_hbm, tbl_v)
    pltpu.sync_copy(sid_hbm.at[pl.ds(base_row, RPW)], sid_v)

    row16 = lax.iota(jnp.int32, LANES)

    for c in range(NCHUNK):
        copies[c].wait()
        for sub in range(SUBG):
            lg = c * SUBG + sub
            sidv = sid_v[pl.ds(lg * LANES, LANES)]
            sb = sidv * 128
            widx0 = row16 * LW + (sub * LANES * LW)

            def body(_, carry):
                acc, cnt, widx = carry
                w = plsc.load_gather(bufs[c], [widx])
                for j in range(4):
                    t = (w >> (8 * j)) if j else w
                    g = plsc.load_gather(tbl_v, [(t & 0x7F) + sb])
                    acc = acc + g
                    cnt = cnt + (t & 0x40)
                return acc, cnt, widx + 1

            acc, cnt, _ = lax.fori_loop(
                0, LW, body,
                (jnp.zeros((LANES,), jnp.float32),
                 jnp.zeros((LANES,), jnp.int32), widx0))

            cnt_f = jnp.maximum((cnt >> 6).astype(jnp.float32), 1.0)
            out_v[pl.ds(lg * LANES, LANES)] = jnp.exp(acc / cnt_f)

    pltpu.sync_copy(out_v, out_hbm.at[pl.ds(base_row, RPW)])


@jax.jit
def kernel(codon_ids, species_ids, mask, weight_matrix):
    logw = jnp.log(jnp.maximum(weight_matrix, 1e-8))
    tbl = jnp.concatenate(
        [jnp.zeros((N_SPECIES, N_CODONS), jnp.float32), logw], axis=1)

    x = codon_ids | (mask.astype(jnp.int32) << 6)
    packed_words = (x[:, 0:LW] | (x[:, LW:2 * LW] << 8)
                    | (x[:, 2 * LW:3 * LW] << 16)
                    | (x[:, 3 * LW:] << 24)).reshape(-1)

    mesh = plsc.VectorSubcoreMesh(core_axis_name="c", subcore_axis_name="s")
    run = pl.kernel(
        _cai_sc,
        mesh=mesh,
        compiler_params=pltpu.CompilerParams(needs_layout_passes=False),
        out_type=jax.ShapeDtypeStruct((B,), jnp.float32),
        scratch_types=[
            pltpu.VMEM((CROWS * LW,), jnp.int32),
            pltpu.VMEM((CROWS * LW,), jnp.int32),
            pltpu.VMEM((CROWS * LW,), jnp.int32),
            pltpu.VMEM((CROWS * LW,), jnp.int32),
            pltpu.VMEM((N_SPECIES * 128,), jnp.float32),
            pltpu.VMEM((RPW,), jnp.int32),
            pltpu.VMEM((RPW,), jnp.float32),
            pltpu.SemaphoreType.DMA,
            pltpu.SemaphoreType.DMA,
            pltpu.SemaphoreType.DMA,
            pltpu.SemaphoreType.DMA,
        ],
    )
    return run(packed_words, species_ids, tbl.reshape(-1))
